# Initial kernel scaffold; baseline (speedup 1.0000x reference)
import jax
import jax.numpy as jnp
from jax.experimental import pallas as pl


def kernel(embedding_input, edge_index, W_fc, W_attn):
    raise NotImplementedError("write your pallas kernel here")


# speedup vs baseline: 2.8954x; 2.8954x over previous
import jax
import jax.numpy as jnp
from jax import lax
from jax.experimental import pallas as pl
from jax.experimental.pallas import tpu as pltpu
from jax.experimental.pallas import tpu_sc as plsc

N = 10000
E = 160000
H = 256
HH = 128
A_ = 1.0
B_ = 1.0
GAMMA = 0.2
NSUB = 16
EP = E // NSUB
EPW = E // 32
KE = 40
NCE = EP // KE
KL = 40
NCL = EPW // KL
KR = 40
NCH = N // KR
F32 = jnp.float32
I32 = jnp.int32


def _mesh():
    return plsc.VectorSubcoreMesh(core_axis_name="c", subcore_axis_name="s",
                                  num_cores=2, num_subcores=NSUB)


_SC_PARAMS = dict(
    mesh=_mesh(),
    compiler_params=pltpu.CompilerParams(needs_layout_passes=False),
)


def _svec_body(ha_ref, hb_ref, wfc_ref, wa_ref, s1_ref, s2_ref):
    wa = wa_ref[...]
    wfc = wfc_ref[...]
    u = jnp.dot(wa[:, :H], wfc, preferred_element_type=F32)
    v = jnp.dot(wa[:, H:], wfc, preferred_element_type=F32)
    ha = ha_ref[...]
    hb = hb_ref[...]
    s1_ref[...] = (jnp.dot(ha, u[:, :HH].T, preferred_element_type=F32)
                   + jnp.dot(hb, u[:, HH:].T, preferred_element_type=F32))
    s2_ref[...] = (jnp.dot(ha, v[:, :HH].T, preferred_element_type=F32)
                   + jnp.dot(hb, v[:, HH:].T, preferred_element_type=F32))


def _tc_svec(ha, hb, W_fc, W_attn):
    out = pl.pallas_call(
        _svec_body,
        out_shape=[jax.ShapeDtypeStruct((N, 1), F32),
                   jax.ShapeDtypeStruct((N, 1), F32)],
    )(ha, hb, W_fc, W_attn)
    return out[0].reshape(N), out[1].reshape(N)


def _sc_layer_body(h2, emb2, s1h, s2h, srch, dsth, srcoffh,
                   h2n, exh, ssumout,
                   ev, gbuf, ebuf, hbuf, zbuf, maxv, maxbv,
                   src80, dst80, dst2row, esb, edb, ssb, sem,
                   aggsh, ssumsh, maxbsh):
    c = lax.axis_index("c")
    t = lax.axis_index("s")
    ebase = t * EP
    lanes = lax.iota(I32, 16)

    def _z(i, _):
        zbuf[pl.ds(i * 16, 16)] = jnp.zeros((16,), F32)
        return 0
    lax.fori_loop(0, 2000 // 16, _z, 0)

    @pl.when(t == 0)
    def _():
        for k in range(N // 2000):
            pltpu.sync_copy(zbuf, ssumsh.at[pl.ds(k * 2000, 2000)])

    def _zh(i, _):
        def _zr(q, _):
            hbuf[i, pl.ds(q * 16, 16)] = jnp.zeros((16,), F32)
            return 0
        lax.fori_loop(0, HH // 16, _zr, 0)
        return 0
    lax.fori_loop(0, KR, _zh, 0)
    for k in range((NCH + NSUB - 1) // NSUB):
        cid = t + k * NSUB
        @pl.when(cid < NCH)
        def _():
            pltpu.sync_copy(hbuf, aggsh.at[pl.ds(cid * KR, KR)])

    def _pa(j, vmax):
        eb = ebase + j * KE
        pltpu.sync_copy(srch.at[pl.ds(eb, KE)], src80)
        pltpu.sync_copy(dsth.at[pl.ds(eb, KE)], dst80)
        pltpu.async_copy(s1h.at[src80], esb, sem).wait()
        pltpu.async_copy(s2h.at[dst80], edb, sem).wait()
        for g0 in (0, 16, 24):
            a = esb[pl.ds(g0, 16)] + edb[pl.ds(g0, 16)]
            e = jnp.maximum(a, GAMMA * a)
            ev[pl.ds(j * KE + g0, 16)] = e
            vmax = jnp.maximum(vmax, e)
        return vmax
    vmax = lax.fori_loop(0, NCE, _pa, jnp.full((16,), -3e38, F32))
    maxv[...] = vmax
    pltpu.sync_copy(maxv, maxbsh.at[t * 8])
    plsc.subcore_barrier()

    pltpu.sync_copy(maxbsh, maxbv)
    vm = maxbv[0]
    for tt in range(1, NSUB):
        vm = jnp.maximum(vm, maxbv[tt * 8])
    gmax = lax.reduce_max(vm, axes=(0,))

    def _pexp(i, _):
        sl = pl.ds(i * 16, 16)
        ev[sl] = jnp.exp(ev[sl] - gmax)
        return 0
    lax.fori_loop(0, EP // 16, _pexp, 0)

    def _pb(j, _):
        pltpu.sync_copy(dsth.at[pl.ds(ebase + j * KE, KE)], dst2row.at[0])
        pltpu.sync_copy(ev.at[pl.ds(j * KE, KE)],
                        ssumsh.at[dst2row.at[0]], add=True)
        return 0
    lax.fori_loop(0, NCE, _pb, 0)
    plsc.subcore_barrier()

    @pl.when(c == 0)
    def _():
        pltpu.sync_copy(ev, exh.at[pl.ds(ebase, EP)])

    @pl.when(jnp.logical_and(c == 0, t < N // 2000))
    def _():
        pltpu.sync_copy(ssumsh.at[pl.ds(t * 2000, 2000)], zbuf)
        pltpu.sync_copy(zbuf, ssumout.at[pl.ds(t * 2000, 2000)])

    coff = c * N
    def _p2(j, _):
        pltpu.sync_copy(srcoffh.at[pl.ds(c * E + ebase + j * KE, KE)], src80)
        pltpu.async_copy(h2.at[src80], gbuf, sem).wait()
        for g0, rlo in ((0, 0), (16, 0), (24, 8)):
            alv = ev[pl.ds(j * KE + g0, 16)]
            def _scale(rr, _):
                al = lax.reduce_sum(jnp.where(lanes == rr, alv, 0.0),
                                    axes=(0,))
                row = g0 + rr
                for q in range(HH // 16):
                    sl = pl.ds(q * 16, 16)
                    gbuf[row, sl] = gbuf[row, sl] * al
                return 0
            lax.fori_loop(rlo, 16, _scale, 0)
        pltpu.sync_copy(dsth.at[pl.ds(ebase + j * KE, KE)], dst2row.at[0])
        pltpu.sync_copy(gbuf, aggsh.at[dst2row.at[0]], add=True)
        return 0
    lax.fori_loop(0, NCE, _p2, 0)
    plsc.subcore_barrier()

    inv = 1.0 / (A_ + B_)
    for k in range((NCH + NSUB - 1) // NSUB):
        cid = t + k * NSUB
        @pl.when(cid < NCH)
        def _():
            r0 = cid * KR
            pltpu.sync_copy(aggsh.at[pl.ds(r0, KR)], hbuf)
            pltpu.sync_copy(emb2.at[pl.ds(coff + r0, KR)], ebuf)
            pltpu.sync_copy(ssumsh.at[pl.ds(r0, KR)], ssb)
            for g0, rlo in ((0, 0), (16, 0), (24, 8)):
                rcpv = (B_ * inv) / jnp.maximum(ssb[pl.ds(g0, 16)], 1e-37)
                def _upd(rr, _):
                    rcp = lax.reduce_sum(
                        jnp.where(lanes == rr, rcpv, 0.0), axes=(0,))
                    row = g0 + rr
                    for q in range(HH // 16):
                        sl = pl.ds(q * 16, 16)
                        hbuf[row, sl] = ((A_ * inv) * ebuf[row, sl]
                                         + rcp * hbuf[row, sl])
                    return 0
                lax.fori_loop(rlo, 16, _upd, 0)
            pltpu.sync_copy(hbuf, h2n.at[pl.ds(coff + r0, KR)])


def _sc_layer(h2, emb2, s1, s2, src, dst, srcoff):
    kfn = pl.kernel(
        _sc_layer_body,
        out_type=[jax.ShapeDtypeStruct((2 * N, HH), F32),
                  jax.ShapeDtypeStruct((E,), F32),
                  jax.ShapeDtypeStruct((N,), F32)],
        scratch_types=[
            pltpu.VMEM((EP,), F32),
            pltpu.VMEM((KE, HH), F32),
            pltpu.VMEM((KR, HH), F32),
            pltpu.VMEM((KR, HH), F32),
            pltpu.VMEM((2000,), F32),
            pltpu.VMEM((16,), F32),
            pltpu.VMEM((NSUB * 8, 16), F32),
            pltpu.VMEM((KE,), I32),
            pltpu.VMEM((KE,), I32),
            pltpu.VMEM((1, KE), I32),
            pltpu.VMEM((KE,), F32),
            pltpu.VMEM((KE,), F32),
            pltpu.VMEM((KR,), F32),
            pltpu.SemaphoreType.DMA,
            pltpu.VMEM_SHARED((N, HH), F32),
            pltpu.VMEM_SHARED((N,), F32),
            pltpu.VMEM_SHARED((NSUB * 8, 16), F32),
        ],
        **_SC_PARAMS,
    )
    return kfn(h2, emb2, s1, s2, src, dst, srcoff)


def _sc_loss_body(h2, srch, dsth, srcNh, dstNh, exhh, ssumh,
                  alphah, d2h,
                  exv, av, d2v, i40a, i40b, i40c, i40d, ssd,
                  gs0, gs1, gd0, gd1, sem):
    c = lax.axis_index("c")
    t = lax.axis_index("s")
    wid = t * 2 + c
    ebase = wid * EPW
    lanes = lax.iota(I32, 16)
    pltpu.sync_copy(exhh.at[pl.ds(ebase, EPW)], exv)

    def _p(j, _):
        sl = pl.ds(j * KL, KL)
        gsl = pl.ds(ebase + j * KL, KL)
        pltpu.sync_copy(srch.at[gsl], i40a)
        pltpu.sync_copy(dsth.at[gsl], i40b)
        pltpu.sync_copy(srcNh.at[gsl], i40c)
        pltpu.sync_copy(dstNh.at[gsl], i40d)
        cps = pltpu.async_copy(ssumh.at[i40b], ssd, sem)
        cp1 = pltpu.async_copy(h2.at[i40a], gs0, sem)
        cp2 = pltpu.async_copy(h2.at[i40b], gd0, sem)
        cp3 = pltpu.async_copy(h2.at[i40c], gs1, sem)
        cp4 = pltpu.async_copy(h2.at[i40d], gd1, sem)
        cps.wait()
        cp1.wait()
        cp2.wait()
        cp3.wait()
        cp4.wait()
        for g0 in (0, 16, 24):
            av[pl.ds(j * KL + g0, 16)] = (exv[pl.ds(j * KL + g0, 16)]
                                          / ssd[pl.ds(g0, 16)])
        for g0, rlo in ((0, 0), (16, 0), (24, 8)):
            def _row(rr, accv):
                r = g0 + rr
                acc = jnp.zeros((16,), F32)
                for q in range(HH // 16):
                    sq = pl.ds(q * 16, 16)
                    d0 = gs0[r, sq] - gd0[r, sq]
                    d1 = gs1[r, sq] - gd1[r, sq]
                    acc = acc + d0 * d0 + d1 * d1
                d2r = lax.reduce_sum(acc, axes=(0,))
                return jnp.where(lanes == rr, d2r, accv)
            accv = lax.fori_loop(rlo, 16, _row, jnp.zeros((16,), F32))
            @pl.when(rlo == 0)
            def _():
                d2v[pl.ds(j * KL + g0, 16)] = accv
            @pl.when(rlo != 0)
            def _():
                old = d2v[pl.ds(j * KL + g0, 16)]
                d2v[pl.ds(j * KL + g0, 16)] = jnp.where(lanes < 8, old, accv)
        return 0
    lax.fori_loop(0, NCL, _p, 0)
    pltpu.sync_copy(av, alphah.at[pl.ds(ebase, EPW)])
    pltpu.sync_copy(d2v, d2h.at[pl.ds(ebase, EPW)])


def _sc_loss(h2, src, dst, srcN, dstN, exh, ssum):
    kfn = pl.kernel(
        _sc_loss_body,
        out_type=[jax.ShapeDtypeStruct((E,), F32),
                  jax.ShapeDtypeStruct((E,), F32)],
        scratch_types=[
            pltpu.VMEM((EPW,), F32),
            pltpu.VMEM((EPW,), F32),
            pltpu.VMEM((EPW,), F32),
            pltpu.VMEM((KL,), I32),
            pltpu.VMEM((KL,), I32),
            pltpu.VMEM((KL,), I32),
            pltpu.VMEM((KL,), I32),
            pltpu.VMEM((KL,), F32),
            pltpu.VMEM((KL, HH), F32),
            pltpu.VMEM((KL, HH), F32),
            pltpu.VMEM((KL, HH), F32),
            pltpu.VMEM((KL, HH), F32),
            pltpu.SemaphoreType.DMA,
        ],
        **_SC_PARAMS,
    )
    return kfn(h2, src, dst, srcN, dstN, exh, ssum)


def _out_body(ha_ref, hb_ref, emb_ref, alpha_ref, d2_ref, h_ref, loss_ref):
    ha = ha_ref[...]
    hb = hb_ref[...]
    emb = emb_ref[...]
    h_ref[:, :HH] = ha
    h_ref[:, HH:] = hb
    d0 = ha - emb[:, :HH]
    d1 = hb - emb[:, HH:]
    la = jnp.sum(d0 * d0) + jnp.sum(d1 * d1)
    dist = jnp.sqrt(d2_ref[...])
    lb = jnp.sum(alpha_ref[...] * dist)
    loss_ref[...] = ((A_ * la + B_ * lb) / N).reshape(1, 1)


def _tc_out(ha, hb, emb, alpha, d2):
    return pl.pallas_call(
        _out_body,
        out_shape=[jax.ShapeDtypeStruct((N, H), F32),
                   jax.ShapeDtypeStruct((1, 1), F32)],
    )(ha, hb, emb, alpha, d2)


def kernel(embedding_input, edge_index, W_fc, W_attn):
    emb = embedding_input
    src = edge_index[0]
    dst = edge_index[1]
    srcN = src + N
    dstN = dst + N
    srcoff = jnp.concatenate([src, srcN])
    ha0 = emb[:, :HH]
    hb0 = emb[:, HH:]
    h2_0 = jnp.concatenate([ha0, hb0], axis=0)

    s1, s2 = _tc_svec(ha0, hb0, W_fc, W_attn)
    h2_1, _, _ = _sc_layer(h2_0, h2_0, s1, s2, src, dst, srcoff)
    s1, s2 = _tc_svec(h2_1[:N], h2_1[N:], W_fc, W_attn)
    h2_2, ex2, ss2 = _sc_layer(h2_1, h2_0, s1, s2, src, dst, srcoff)
    alpha, d2 = _sc_loss(h2_2, src, dst, srcN, dstN, ex2, ss2)
    h_out, loss = _tc_out(h2_2[:N], h2_2[N:], emb, alpha, d2)
    return h_out, loss[0, 0]
